# Initial kernel scaffold; baseline (speedup 1.0000x reference)
import jax
import jax.numpy as jnp
from jax.experimental import pallas as pl


def kernel(x, W):
    raise NotImplementedError("write your pallas kernel here")


# speedup vs baseline: 3.5997x; 3.5997x over previous
import numpy as np
import jax
import jax.numpy as jnp
from jax import lax
from jax.experimental import pallas as pl
from jax.experimental.pallas import tpu as pltpu
from jax.experimental.pallas import tpu_sc as plsc

_F = 26
_VOCAB = 26000
_D = 64
_B = 4096
_NC, _NS, _L = 2, 16, 16
_NW = _NC * _NS
_BPW = _B // _NW
_NROW = _F * _F
_NCHUNK = 6
_NROW_PAD = _NCHUNK * 128


def _build_consts():
    e = np.arange(_NROW_PAD)
    m = np.where(e < _NROW, e // _F, 0)
    k = np.where(e < _NROW, e % _F, 0)
    base = (m * _VOCAB + k * 1000).astype(np.int32)
    kmod = k.astype(np.int32)
    return base, kmod


_BASE_NP, _KMOD_NP = _build_consts()


def _ffm_body(x_hbm, w_hbm, base_hbm, kmod_hbm, out_hbm,
              xs_v, base_v, kmod_v, idx_v, rows_v, part_v, out_v, sem):
    wid = lax.axis_index("s") * _NC + lax.axis_index("c")
    sbase = wid * _BPW

    pltpu.sync_copy(x_hbm.at[pl.ds(sbase * _F, _BPW * _F)], xs_v)
    pltpu.sync_copy(base_hbm, base_v)
    pltpu.sync_copy(kmod_hbm, kmod_v)

    def sample_body(s, carry):
        for c in range(_NROW_PAD // _L):
            sl = pl.ds(c * _L, _L)
            km = kmod_v[sl]
            xv = plsc.load_gather(xs_v, [s * _F + km])
            idx_v[c // 8, pl.ds((c % 8) * _L, _L)] = base_v[sl] + xv

        copies = [
            pltpu.async_copy(
                w_hbm.at[idx_v.at[c]],
                rows_v.at[pl.ds(c * 128, 128), :],
                sem,
            )
            for c in range(_NCHUNK)
        ]
        for cp in copies:
            cp.wait()

        def i_body(i, acc):
            def j_body(j, acc):
                a = j * _F + i
                b = i * _F + j
                return tuple(
                    acc[q]
                    + rows_v[a, pl.ds(q * _L, _L)] * rows_v[b, pl.ds(q * _L, _L)]
                    for q in range(_D // _L)
                )
            return lax.fori_loop(i + 1, _F, j_body, acc)

        zero = jnp.zeros((_L,), jnp.float32)
        accs = lax.fori_loop(0, _F - 1, i_body, (zero, zero, zero, zero))
        tot = accs[0] + accs[1] + accs[2] + accs[3]
        part_v[pl.ds(s * _L, _L)] = tot
        return carry

    lax.fori_loop(0, _BPW, sample_body, 0)

    lanes = lax.iota(jnp.int32, _L)
    for g in range(_BPW // _L):
        acc = jnp.zeros((_L,), jnp.float32)
        for c in range(_L):
            gidx = g * _L * _L + lanes * _L + c
            acc = acc + plsc.load_gather(part_v, [gidx])
        out_v[pl.ds(g * _L, _L)] = acc

    pltpu.sync_copy(out_v, out_hbm.at[pl.ds(sbase, _BPW)])


@jax.jit
def kernel(x, W):
    xi = x.astype(jnp.int32).reshape(_B * _F)
    wf = W.reshape(_F * _VOCAB, _D)
    mesh = plsc.VectorSubcoreMesh(
        core_axis_name="c", subcore_axis_name="s",
        num_cores=_NC, num_subcores=_NS,
    )
    run = pl.kernel(
        _ffm_body,
        out_type=jax.ShapeDtypeStruct((_B,), jnp.float32),
        mesh=mesh,
        compiler_params=pltpu.CompilerParams(
            needs_layout_passes=False, use_tc_tiling_on_sc=False,
        ),
        scratch_types=[
            pltpu.VMEM((_BPW * _F,), jnp.int32),
            pltpu.VMEM((_NROW_PAD,), jnp.int32),
            pltpu.VMEM((_NROW_PAD,), jnp.int32),
            pltpu.VMEM((_NCHUNK, 128), jnp.int32),
            pltpu.VMEM((_NROW_PAD, _D), jnp.float32),
            pltpu.VMEM((_BPW * _L,), jnp.float32),
            pltpu.VMEM((_BPW,), jnp.float32),
            pltpu.SemaphoreType.DMA,
        ],
    )
    out = run(xi, wf, jnp.asarray(_BASE_NP), jnp.asarray(_KMOD_NP))
    return out[:, None]
